# baseline (speedup 1.0000x reference)
import functools

import jax
import jax.numpy as jnp
from jax import lax
from jax.experimental import pallas as pl
from jax.experimental.pallas import tpu as pltpu
from jax.experimental.pallas import tpu_sc as plsc

_BASE_VOCAB = 1000000
_EMBED_DIM = 64
_NUM_SEGMENTS = 250000
_N = 1000000

_NW = 32
_SEG_TILE = 400
_NTILES = _NUM_SEGMENTS // _SEG_TILE
_PROWS = _SEG_TILE // 2
_CHUNK = 128
_TILES_PER_W = (_NTILES + _NW - 1) // _NW
_STARTS_PAD = 648
_NBUF = 4
_CNT_ROWS = (_SEG_TILE + 1 + 7) // 8 + 1


def _sc_kernel(table, eidx, sidx, starts, out, starts_v, idx_v, seg_v,
               idxh_v, half_v, rows_v, acc, cnt, isem, ssem, gsem):
    wid = lax.axis_index("s") * 2 + lax.axis_index("c")
    pltpu.sync_copy(starts.at[pl.ds(0, _STARTS_PAD)], starts_v)

    ones = jnp.full((16,), 1.0, dtype=jnp.float32)
    zeros = jnp.zeros((16,), dtype=jnp.float32)

    def tile_body(i, _):
        t = wid + i * _NW

        @pl.when(t < _NTILES)
        def _():
            sv = starts_v[pl.ds(t, 16)]
            r_lo = sv[0]
            r_hi = sv[1]
            r0 = lax.bitwise_and(r_lo, jnp.int32(-8))
            nchunks = (r_hi - r0 + (_CHUNK - 1)) // _CHUNK
            seg_lo = t * _SEG_TILE

            def chunk_base(k):
                return pl.multiple_of(
                    jnp.minimum(r0 + k * _CHUNK, _N - _CHUNK), 8)

            def start_idx(k):
                s = lax.rem(k, _NBUF)
                b = chunk_base(k)
                pltpu.async_copy(eidx.at[pl.ds(b, _CHUNK)], idx_v.at[s],
                                 isem.at[s])
                pltpu.async_copy(sidx.at[pl.ds(b, _CHUNK)], seg_v.at[s],
                                 ssem.at[s])

            def start_gather(k):
                s = lax.rem(k, _NBUF)
                s2 = lax.rem(k, 2)
                b = chunk_base(k)
                pltpu.make_async_copy(eidx.at[pl.ds(b, _CHUNK)],
                                      idx_v.at[s], isem.at[s]).wait()
                for g in range(_CHUNK // 16):
                    v = idx_v[s, pl.ds(g * 16, 16)]
                    idxh_v[s, pl.ds(g * 16, 16)] = lax.shift_right_logical(
                        v, 1)
                    half_v[s, pl.ds(g * 16, 16)] = lax.shift_left(
                        lax.bitwise_and(v, 1), 6)
                pltpu.async_copy(table.at[idxh_v.at[s]], rows_v.at[s2],
                                 gsem.at[s2])

            @pl.when(nchunks > 0)
            def _():
                start_idx(0)

                @pl.when(nchunks > 1)
                def _():
                    start_idx(1)

            def zero_body(p, _):
                for c in range(8):
                    acc[p, pl.ds(c * 16, 16)] = zeros
                return 0

            lax.fori_loop(0, _PROWS + 1, zero_body, 0)

            def zero_cnt(p, _):
                for c in range(8):
                    cnt[p, pl.ds(c * 16, 16)] = zeros
                return 0

            lax.fori_loop(0, _CNT_ROWS, zero_cnt, 0)

            @pl.when(nchunks > 0)
            def _():
                start_gather(0)

            def chunk_body(k, _):
                s = lax.rem(k, _NBUF)
                s2 = lax.rem(k, 2)
                b = chunk_base(k)
                shift = (r0 + k * _CHUNK) - b

                @pl.when(k + 1 < nchunks)
                def _():
                    start_gather(k + 1)

                @pl.when(k + 2 < nchunks)
                def _():
                    start_idx(k + 2)

                pltpu.make_async_copy(table.at[idxh_v.at[s]], rows_v.at[s2],
                                      gsem.at[s2]).wait()
                pltpu.make_async_copy(sidx.at[pl.ds(b, _CHUNK)],
                                      seg_v.at[s], ssem.at[s]).wait()

                def grp_body(g, _):
                    segs = seg_v[s, pl.ds(g * 16, 16)]
                    halfs = half_v[s, pl.ds(g * 16, 16)]
                    for lane in range(16):
                        seg = segs[lane]
                        hoff = halfs[lane]
                        ls = seg - seg_lo
                        j = g * 16 + lane
                        lsc = jnp.where(
                            (ls >= 0) & (ls < _SEG_TILE) & (j >= shift),
                            ls, _SEG_TILE)
                        pr = lax.shift_right_logical(lsc, 1)
                        pc = lax.shift_left(lax.bitwise_and(lsc, 1), 6)
                        for c in range(4):
                            plsc.addupdate(
                                acc.at[pr, pl.ds(pc + c * 16, 16)],
                                rows_v[s2, j, pl.ds(hoff + c * 16, 16)])
                        plsc.addupdate(
                            cnt.at[lax.shift_right_logical(lsc, 3),
                                   pl.ds(lax.shift_left(
                                       lax.bitwise_and(lsc, 7), 4), 16)],
                            ones)
                    return 0

                lax.fori_loop(0, _CHUNK // 16, grp_body, 0)
                return 0

            lax.fori_loop(0, nchunks, chunk_body, 0)

            def fin_body(p, _):
                for half in range(2):
                    sidx_l = 2 * p + half
                    cv = cnt[lax.shift_right_logical(sidx_l, 3),
                             pl.ds(lax.shift_left(
                                 lax.bitwise_and(sidx_l, 7), 4), 16)]
                    inv = 1.0 / jnp.maximum(cv, 1.0)
                    for c in range(4):
                        acc[p, pl.ds(half * 64 + c * 16, 16)] *= inv
                return 0

            lax.fori_loop(0, _PROWS, fin_body, 0)
            pltpu.sync_copy(acc.at[pl.ds(0, _PROWS)],
                            out.at[pl.ds(t * _PROWS, _PROWS)])

        return 0

    lax.fori_loop(0, _TILES_PER_W, tile_body, 0)


def kernel(base_embeddings, extract_index, scatter_index):
    bounds = jnp.arange(0, _STARTS_PAD * _SEG_TILE, _SEG_TILE,
                        dtype=jnp.int32)
    starts = jnp.searchsorted(scatter_index, bounds).astype(jnp.int32)
    table = base_embeddings.reshape(_BASE_VOCAB // 2, 2 * _EMBED_DIM)

    mesh = plsc.VectorSubcoreMesh(core_axis_name="c", subcore_axis_name="s")
    f = functools.partial(
        pl.kernel,
        mesh=mesh,
        out_type=jax.ShapeDtypeStruct((_NUM_SEGMENTS // 2, 2 * _EMBED_DIM),
                                      jnp.float32),
        scratch_types=[
            pltpu.VMEM((_STARTS_PAD,), jnp.int32),
            pltpu.VMEM((_NBUF, _CHUNK), jnp.int32),
            pltpu.VMEM((_NBUF, _CHUNK), jnp.int32),
            pltpu.VMEM((_NBUF, _CHUNK), jnp.int32),
            pltpu.VMEM((_NBUF, _CHUNK), jnp.int32),
            pltpu.VMEM((2, _CHUNK, 2 * _EMBED_DIM), jnp.float32),
            pltpu.VMEM((_PROWS + 1, 2 * _EMBED_DIM), jnp.float32),
            pltpu.VMEM((_CNT_ROWS, 128), jnp.float32),
            pltpu.SemaphoreType.DMA((_NBUF,)),
            pltpu.SemaphoreType.DMA((_NBUF,)),
            pltpu.SemaphoreType.DMA((2,)),
        ],
        compiler_params=pltpu.CompilerParams(use_tc_tiling_on_sc=True),
    )(_sc_kernel)
    out = f(table, extract_index, scatter_index, starts)
    return out.reshape(_NUM_SEGMENTS, _EMBED_DIM)


# speedup vs baseline: 1.2145x; 1.2145x over previous
import functools

import jax
import jax.numpy as jnp
from jax import lax
from jax.experimental import pallas as pl
from jax.experimental.pallas import tpu as pltpu
from jax.experimental.pallas import tpu_sc as plsc

_BASE_VOCAB = 1000000
_EMBED_DIM = 64
_NUM_SEGMENTS = 250000
_N = 1000000

_NW = 32
_NSUB = 16
_SEG_TILE = 400
_NTILES = _NUM_SEGMENTS // _SEG_TILE
_CHUNK = 128
_TILES_PER_W = (_NTILES + _NW - 1) // _NW
_STARTS_PAD = 648
_NBUF = 4
_ACC_STRIDE = 416


def _sc_kernel(table, eidx, sidx, starts, out, starts_v, idx_v, seg_v,
               segloc_v, rows_v, accv, cntv, zpad, zpadc, ones_b,
               acc_sh, cnt_sh, isem, ssem, gsem, zsem):
    wid = lax.axis_index("s") * 2 + lax.axis_index("c")
    base_row = lax.axis_index("s") * _ACC_STRIDE
    pltpu.sync_copy(starts.at[pl.ds(0, _STARTS_PAD)], starts_v)

    zeros = jnp.zeros((16,), dtype=jnp.float32)
    onesv = jnp.full((16,), 1.0, dtype=jnp.float32)

    def init_body(p, _):
        for c in range(4):
            zpad[p, pl.ds(c * 16, 16)] = zeros
        zpadc[p, :] = zeros
        ones_b[p, :] = onesv
        return 0

    lax.fori_loop(0, _CHUNK, init_body, 0)

    _ZFULL = 3
    _ZREM = 17

    def zero_acc():
        for p in range(_ZFULL):
            pltpu.async_copy(
                zpad.at[pl.ds(0, _CHUNK)],
                acc_sh.at[pl.ds(base_row + p * _CHUNK, _CHUNK)], zsem)
            pltpu.async_copy(
                zpadc.at[pl.ds(0, _CHUNK)],
                cnt_sh.at[pl.ds(base_row + p * _CHUNK, _CHUNK)], zsem)
        pltpu.async_copy(zpad.at[pl.ds(0, _ZREM)],
                         acc_sh.at[pl.ds(base_row + _ZFULL * _CHUNK, _ZREM)],
                         zsem)
        pltpu.async_copy(zpadc.at[pl.ds(0, _ZREM)],
                         cnt_sh.at[pl.ds(base_row + _ZFULL * _CHUNK, _ZREM)],
                         zsem)
        for p in range(_ZFULL):
            pltpu.make_async_copy(
                zpad.at[pl.ds(0, _CHUNK)],
                acc_sh.at[pl.ds(base_row + p * _CHUNK, _CHUNK)],
                zsem).wait()
            pltpu.make_async_copy(
                zpadc.at[pl.ds(0, _CHUNK)],
                cnt_sh.at[pl.ds(base_row + p * _CHUNK, _CHUNK)],
                zsem).wait()
        pltpu.make_async_copy(zpad.at[pl.ds(0, _ZREM)],
                              acc_sh.at[pl.ds(base_row + _ZFULL * _CHUNK, _ZREM)],
                              zsem).wait()
        pltpu.make_async_copy(zpadc.at[pl.ds(0, _ZREM)],
                              cnt_sh.at[pl.ds(base_row + _ZFULL * _CHUNK, _ZREM)],
                              zsem).wait()

    def tile_body(i, _):
        t = wid + i * _NW

        @pl.when(t < _NTILES)
        def _():
            sv = starts_v[pl.ds(t, 16)]
            r_lo = sv[0]
            r_hi = sv[1]
            r0 = lax.bitwise_and(r_lo, jnp.int32(-8))
            nchunks = (r_hi - r0 + (_CHUNK - 1)) // _CHUNK
            seg_lo = t * _SEG_TILE

            def chunk_base(k):
                return pl.multiple_of(
                    jnp.minimum(r0 + k * _CHUNK, _N - _CHUNK), 8)

            def start_idx(k):
                s = lax.rem(k, _NBUF)
                b = chunk_base(k)
                pltpu.async_copy(eidx.at[pl.ds(b, _CHUNK)], idx_v.at[s],
                                 isem.at[s])
                pltpu.async_copy(sidx.at[pl.ds(b, _CHUNK)], seg_v.at[s],
                                 ssem.at[s])

            def start_gather(k):
                s = lax.rem(k, _NBUF)
                s2 = lax.rem(k, 2)
                b = chunk_base(k)
                pltpu.make_async_copy(eidx.at[pl.ds(b, _CHUNK)],
                                      idx_v.at[s], isem.at[s]).wait()
                pltpu.async_copy(table.at[idx_v.at[s]], rows_v.at[s2],
                                 gsem.at[s2])

            @pl.when(nchunks > 0)
            def _():
                start_idx(0)

                @pl.when(nchunks > 1)
                def _():
                    start_idx(1)

            zero_acc()

            @pl.when(nchunks > 0)
            def _():
                start_gather(0)

            def chunk_body(k, _):
                s = lax.rem(k, _NBUF)
                s2 = lax.rem(k, 2)
                b = chunk_base(k)
                shift = (r0 + k * _CHUNK) - b

                @pl.when(k + 1 < nchunks)
                def _():
                    start_gather(k + 1)

                @pl.when(k + 2 < nchunks)
                def _():
                    start_idx(k + 2)

                pltpu.make_async_copy(sidx.at[pl.ds(b, _CHUNK)],
                                      seg_v.at[s], ssem.at[s]).wait()
                for g in range(_CHUNK // 16):
                    sv16 = seg_v[s, pl.ds(g * 16, 16)]
                    ls = sv16 - seg_lo
                    jv = lax.iota(jnp.int32, 16) + (g * 16)
                    lsc = jnp.where(
                        (ls >= 0) & (ls < _SEG_TILE) & (jv >= shift),
                        ls, _SEG_TILE)
                    segloc_v[s, pl.ds(g * 16, 16)] = lsc + base_row

                pltpu.make_async_copy(table.at[idx_v.at[s]], rows_v.at[s2],
                                      gsem.at[s2]).wait()
                pltpu.sync_copy(rows_v.at[s2],
                                acc_sh.at[segloc_v.at[s]], add=True)
                pltpu.sync_copy(ones_b.at[pl.ds(0, _CHUNK)],
                                cnt_sh.at[segloc_v.at[s]], add=True)
                return 0

            lax.fori_loop(0, nchunks, chunk_body, 0)

            pltpu.sync_copy(acc_sh.at[pl.ds(base_row, _SEG_TILE)], accv)
            pltpu.sync_copy(cnt_sh.at[pl.ds(base_row, _SEG_TILE)], cntv)

            def fin_body(p, _):
                inv = 1.0 / jnp.maximum(cntv[p, :], 1.0)
                for c in range(4):
                    accv[p, pl.ds(c * 16, 16)] *= inv
                return 0

            lax.fori_loop(0, _SEG_TILE, fin_body, 0)
            pltpu.sync_copy(accv.at[pl.ds(0, _SEG_TILE)],
                            out.at[pl.ds(seg_lo, _SEG_TILE)])

        return 0

    lax.fori_loop(0, _TILES_PER_W, tile_body, 0)


def kernel(base_embeddings, extract_index, scatter_index):
    bounds = jnp.arange(0, _STARTS_PAD * _SEG_TILE, _SEG_TILE,
                        dtype=jnp.int32)
    starts = jnp.searchsorted(scatter_index, bounds).astype(jnp.int32)

    mesh = plsc.VectorSubcoreMesh(core_axis_name="c", subcore_axis_name="s")
    f = functools.partial(
        pl.kernel,
        mesh=mesh,
        out_type=jax.ShapeDtypeStruct((_NUM_SEGMENTS, _EMBED_DIM),
                                      jnp.float32),
        scratch_types=[
            pltpu.VMEM((_STARTS_PAD,), jnp.int32),
            pltpu.VMEM((_NBUF, _CHUNK), jnp.int32),
            pltpu.VMEM((_NBUF, _CHUNK), jnp.int32),
            pltpu.VMEM((_NBUF, _CHUNK), jnp.int32),
            pltpu.VMEM((2, _CHUNK, _EMBED_DIM), jnp.float32),
            pltpu.VMEM((_SEG_TILE, _EMBED_DIM), jnp.float32),
            pltpu.VMEM((_SEG_TILE, 16), jnp.float32),
            pltpu.VMEM((_CHUNK, _EMBED_DIM), jnp.float32),
            pltpu.VMEM((_CHUNK, 16), jnp.float32),
            pltpu.VMEM((_CHUNK, 16), jnp.float32),
            pltpu.VMEM_SHARED((_NSUB * _ACC_STRIDE, _EMBED_DIM),
                              jnp.float32),
            pltpu.VMEM_SHARED((_NSUB * _ACC_STRIDE, 16), jnp.float32),
            pltpu.SemaphoreType.DMA((_NBUF,)),
            pltpu.SemaphoreType.DMA((_NBUF,)),
            pltpu.SemaphoreType.DMA((2,)),
            pltpu.SemaphoreType.DMA,
        ],
        compiler_params=pltpu.CompilerParams(use_tc_tiling_on_sc=False),
    )(_sc_kernel)
    return f(base_embeddings, extract_index, scatter_index, starts)


# speedup vs baseline: 1.2208x; 1.0052x over previous
import functools

import jax
import jax.numpy as jnp
from jax import lax
from jax.experimental import pallas as pl
from jax.experimental.pallas import tpu as pltpu
from jax.experimental.pallas import tpu_sc as plsc

_BASE_VOCAB = 1000000
_EMBED_DIM = 64
_NUM_SEGMENTS = 250000
_N = 1000000

_NW = 32
_NSUB = 16
_SEG_TILE = 400
_NTILES = _NUM_SEGMENTS // _SEG_TILE
_CHUNK = 128
_TILES_PER_W = (_NTILES + _NW - 1) // _NW
_STARTS_PAD = 648
_NBUF = 4
_ACC_STRIDE = 416


def _sc_kernel(table, eidx, sidx, starts, out, starts_v, idx_v, seg_v,
               segloc_v, rows_v, accv, cntv, zpad, zpadc, ones_b,
               acc_sh, cnt_sh, isem, ssem, gsem, csem, zsem):
    wid = lax.axis_index("s") * 2 + lax.axis_index("c")
    base_row = lax.axis_index("s") * _ACC_STRIDE
    pltpu.sync_copy(starts.at[pl.ds(0, _STARTS_PAD)], starts_v)

    zeros = jnp.zeros((16,), dtype=jnp.float32)
    onesv = jnp.full((16,), 1.0, dtype=jnp.float32)

    def init_body(p, _):
        for c in range(4):
            zpad[p, pl.ds(c * 16, 16)] = zeros
        zpadc[p, :] = zeros
        ones_b[p, :] = onesv
        return 0

    lax.fori_loop(0, _CHUNK, init_body, 0)

    _ZFULL = 3
    _ZREM = 17

    def zero_acc():
        for p in range(_ZFULL):
            pltpu.async_copy(
                zpad.at[pl.ds(0, _CHUNK)],
                acc_sh.at[pl.ds(base_row + p * _CHUNK, _CHUNK)], zsem)
            pltpu.async_copy(
                zpadc.at[pl.ds(0, _CHUNK)],
                cnt_sh.at[pl.ds(base_row + p * _CHUNK, _CHUNK)], zsem)
        pltpu.async_copy(zpad.at[pl.ds(0, _ZREM)],
                         acc_sh.at[pl.ds(base_row + _ZFULL * _CHUNK, _ZREM)],
                         zsem)
        pltpu.async_copy(zpadc.at[pl.ds(0, _ZREM)],
                         cnt_sh.at[pl.ds(base_row + _ZFULL * _CHUNK, _ZREM)],
                         zsem)
        for p in range(_ZFULL):
            pltpu.make_async_copy(
                zpad.at[pl.ds(0, _CHUNK)],
                acc_sh.at[pl.ds(base_row + p * _CHUNK, _CHUNK)],
                zsem).wait()
            pltpu.make_async_copy(
                zpadc.at[pl.ds(0, _CHUNK)],
                cnt_sh.at[pl.ds(base_row + p * _CHUNK, _CHUNK)],
                zsem).wait()
        pltpu.make_async_copy(zpad.at[pl.ds(0, _ZREM)],
                              acc_sh.at[pl.ds(base_row + _ZFULL * _CHUNK, _ZREM)],
                              zsem).wait()
        pltpu.make_async_copy(zpadc.at[pl.ds(0, _ZREM)],
                              cnt_sh.at[pl.ds(base_row + _ZFULL * _CHUNK, _ZREM)],
                              zsem).wait()

    def tile_body(i, _):
        t = wid + i * _NW

        @pl.when(t < _NTILES)
        def _():
            sv = starts_v[pl.ds(t, 16)]
            r_lo = sv[0]
            r_hi = sv[1]
            r0 = lax.bitwise_and(r_lo, jnp.int32(-8))
            nchunks = (r_hi - r0 + (_CHUNK - 1)) // _CHUNK
            seg_lo = t * _SEG_TILE

            def chunk_base(k):
                return pl.multiple_of(
                    jnp.minimum(r0 + k * _CHUNK, _N - _CHUNK), 8)

            def start_idx(k):
                s = lax.rem(k, _NBUF)
                b = chunk_base(k)
                pltpu.async_copy(eidx.at[pl.ds(b, _CHUNK)], idx_v.at[s],
                                 isem.at[s])
                pltpu.async_copy(sidx.at[pl.ds(b, _CHUNK)], seg_v.at[s],
                                 ssem.at[s])

            def wait_scatter(k):
                s = lax.rem(k, _NBUF)
                s2 = lax.rem(k, 2)
                pltpu.make_async_copy(rows_v.at[s2],
                                      acc_sh.at[segloc_v.at[s]],
                                      csem.at[s2]).wait()
                pltpu.make_async_copy(ones_b.at[pl.ds(0, _CHUNK)],
                                      cnt_sh.at[segloc_v.at[s]],
                                      csem.at[s2]).wait()

            def start_gather(k):
                s = lax.rem(k, _NBUF)
                s2 = lax.rem(k, 2)
                b = chunk_base(k)

                @pl.when(k >= 2)
                def _():
                    wait_scatter(k - 2)

                pltpu.make_async_copy(eidx.at[pl.ds(b, _CHUNK)],
                                      idx_v.at[s], isem.at[s]).wait()
                pltpu.async_copy(table.at[idx_v.at[s]], rows_v.at[s2],
                                 gsem.at[s2])

            @pl.when(nchunks > 0)
            def _():
                start_idx(0)

                @pl.when(nchunks > 1)
                def _():
                    start_idx(1)

            zero_acc()

            @pl.when(nchunks > 0)
            def _():
                start_gather(0)

            def chunk_body(k, _):
                s = lax.rem(k, _NBUF)
                s2 = lax.rem(k, 2)
                b = chunk_base(k)
                shift = (r0 + k * _CHUNK) - b

                @pl.when(k + 1 < nchunks)
                def _():
                    start_gather(k + 1)

                @pl.when(k + 2 < nchunks)
                def _():
                    start_idx(k + 2)

                pltpu.make_async_copy(sidx.at[pl.ds(b, _CHUNK)],
                                      seg_v.at[s], ssem.at[s]).wait()
                for g in range(_CHUNK // 16):
                    sv16 = seg_v[s, pl.ds(g * 16, 16)]
                    ls = sv16 - seg_lo
                    jv = lax.iota(jnp.int32, 16) + (g * 16)
                    lsc = jnp.where(
                        (ls >= 0) & (ls < _SEG_TILE) & (jv >= shift),
                        ls, _SEG_TILE)
                    segloc_v[s, pl.ds(g * 16, 16)] = lsc + base_row

                pltpu.make_async_copy(table.at[idx_v.at[s]], rows_v.at[s2],
                                      gsem.at[s2]).wait()
                pltpu.async_copy(rows_v.at[s2],
                                acc_sh.at[segloc_v.at[s]], csem.at[s2],
                                add=True)
                pltpu.async_copy(ones_b.at[pl.ds(0, _CHUNK)],
                                cnt_sh.at[segloc_v.at[s]], csem.at[s2],
                                add=True)
                return 0

            lax.fori_loop(0, nchunks, chunk_body, 0)

            @pl.when(nchunks > 1)
            def _():
                wait_scatter(nchunks - 2)

            @pl.when(nchunks > 0)
            def _():
                wait_scatter(nchunks - 1)

            pltpu.sync_copy(acc_sh.at[pl.ds(base_row, _SEG_TILE)], accv)
            pltpu.sync_copy(cnt_sh.at[pl.ds(base_row, _SEG_TILE)], cntv)

            def fin_body(p, _):
                inv = 1.0 / jnp.maximum(cntv[p, :], 1.0)
                for c in range(4):
                    accv[p, pl.ds(c * 16, 16)] *= inv
                return 0

            lax.fori_loop(0, _SEG_TILE, fin_body, 0)
            pltpu.sync_copy(accv.at[pl.ds(0, _SEG_TILE)],
                            out.at[pl.ds(seg_lo, _SEG_TILE)])

        return 0

    lax.fori_loop(0, _TILES_PER_W, tile_body, 0)


def kernel(base_embeddings, extract_index, scatter_index):
    bounds = jnp.arange(0, _STARTS_PAD * _SEG_TILE, _SEG_TILE,
                        dtype=jnp.int32)
    starts = jnp.searchsorted(scatter_index, bounds).astype(jnp.int32)

    mesh = plsc.VectorSubcoreMesh(core_axis_name="c", subcore_axis_name="s")
    f = functools.partial(
        pl.kernel,
        mesh=mesh,
        out_type=jax.ShapeDtypeStruct((_NUM_SEGMENTS, _EMBED_DIM),
                                      jnp.float32),
        scratch_types=[
            pltpu.VMEM((_STARTS_PAD,), jnp.int32),
            pltpu.VMEM((_NBUF, _CHUNK), jnp.int32),
            pltpu.VMEM((_NBUF, _CHUNK), jnp.int32),
            pltpu.VMEM((_NBUF, _CHUNK), jnp.int32),
            pltpu.VMEM((2, _CHUNK, _EMBED_DIM), jnp.float32),
            pltpu.VMEM((_SEG_TILE, _EMBED_DIM), jnp.float32),
            pltpu.VMEM((_SEG_TILE, 16), jnp.float32),
            pltpu.VMEM((_CHUNK, _EMBED_DIM), jnp.float32),
            pltpu.VMEM((_CHUNK, 16), jnp.float32),
            pltpu.VMEM((_CHUNK, 16), jnp.float32),
            pltpu.VMEM_SHARED((_NSUB * _ACC_STRIDE, _EMBED_DIM),
                              jnp.float32),
            pltpu.VMEM_SHARED((_NSUB * _ACC_STRIDE, 16), jnp.float32),
            pltpu.SemaphoreType.DMA((_NBUF,)),
            pltpu.SemaphoreType.DMA((_NBUF,)),
            pltpu.SemaphoreType.DMA((2,)),
            pltpu.SemaphoreType.DMA((2,)),
            pltpu.SemaphoreType.DMA,
        ],
        compiler_params=pltpu.CompilerParams(use_tc_tiling_on_sc=False),
    )(_sc_kernel)
    return f(base_embeddings, extract_index, scatter_index, starts)


# speedup vs baseline: 1.2456x; 1.0202x over previous
import functools

import jax
import jax.numpy as jnp
from jax import lax
from jax.experimental import pallas as pl
from jax.experimental.pallas import tpu as pltpu
from jax.experimental.pallas import tpu_sc as plsc

_BASE_VOCAB = 1000000
_EMBED_DIM = 64
_NUM_SEGMENTS = 250000
_N = 1000000

_NW = 32
_NSUB = 16
_SEG_TILE = 400
_NTILES = _NUM_SEGMENTS // _SEG_TILE
_CHUNK = 128
_TILES_PER_W = (_NTILES + _NW - 1) // _NW
_STARTS_PAD = 648
_NBUF = 4
_ACC_STRIDE = 416


def _sc_kernel(table, eidx, sidx, starts, out, starts_v, idx_v, seg_v,
               segloc_v, rows_v, accv, cntv, zpad, zpadc, ones_b,
               acc_sh, cnt_sh, isem, ssem, gsem, csem, zsem):
    wid = lax.axis_index("s") * 2 + lax.axis_index("c")
    region0 = lax.axis_index("s") * (2 * _ACC_STRIDE)
    pltpu.sync_copy(starts.at[pl.ds(0, _STARTS_PAD)], starts_v)

    zeros = jnp.zeros((16,), dtype=jnp.float32)
    onesv = jnp.full((16,), 1.0, dtype=jnp.float32)

    def init_body(p, _):
        for c in range(4):
            zpad[p, pl.ds(c * 16, 16)] = zeros
        zpadc[p, :] = zeros
        ones_b[p, :] = onesv
        return 0

    lax.fori_loop(0, _CHUNK, init_body, 0)

    _ZFULL = 3
    _ZREM = 17

    def fire_zero(base_row):
        for p in range(_ZFULL):
            pltpu.async_copy(
                zpad.at[pl.ds(0, _CHUNK)],
                acc_sh.at[pl.ds(base_row + p * _CHUNK, _CHUNK)], zsem)
            pltpu.async_copy(
                zpadc.at[pl.ds(0, _CHUNK)],
                cnt_sh.at[pl.ds(base_row + p * _CHUNK, _CHUNK)], zsem)
        pltpu.async_copy(zpad.at[pl.ds(0, _ZREM)],
                         acc_sh.at[pl.ds(base_row + _ZFULL * _CHUNK, _ZREM)],
                         zsem)
        pltpu.async_copy(zpadc.at[pl.ds(0, _ZREM)],
                         cnt_sh.at[pl.ds(base_row + _ZFULL * _CHUNK, _ZREM)],
                         zsem)

    def drain_zero(base_row):
        for p in range(_ZFULL):
            pltpu.make_async_copy(
                zpad.at[pl.ds(0, _CHUNK)],
                acc_sh.at[pl.ds(base_row + p * _CHUNK, _CHUNK)],
                zsem).wait()
            pltpu.make_async_copy(
                zpadc.at[pl.ds(0, _CHUNK)],
                cnt_sh.at[pl.ds(base_row + p * _CHUNK, _CHUNK)],
                zsem).wait()
        pltpu.make_async_copy(zpad.at[pl.ds(0, _ZREM)],
                              acc_sh.at[pl.ds(base_row + _ZFULL * _CHUNK, _ZREM)],
                              zsem).wait()
        pltpu.make_async_copy(zpadc.at[pl.ds(0, _ZREM)],
                              cnt_sh.at[pl.ds(base_row + _ZFULL * _CHUNK, _ZREM)],
                              zsem).wait()

    def finalize(tp, base_row):
        pltpu.sync_copy(acc_sh.at[pl.ds(base_row, _SEG_TILE)], accv)
        pltpu.sync_copy(cnt_sh.at[pl.ds(base_row, _SEG_TILE)], cntv)

        def fin_body(p, _):
            inv = 1.0 / jnp.maximum(cntv[p, :], 1.0)
            for c in range(4):
                accv[p, pl.ds(c * 16, 16)] *= inv
            return 0

        lax.fori_loop(0, _SEG_TILE, fin_body, 0)
        pltpu.sync_copy(accv.at[pl.ds(0, _SEG_TILE)],
                        out.at[pl.ds(tp * _SEG_TILE, _SEG_TILE)])

    def tile_body(i, _):
        t = wid + i * _NW
        base_row = region0 + lax.rem(i, 2) * _ACC_STRIDE
        prev_base = region0 + lax.rem(i + 1, 2) * _ACC_STRIDE

        @pl.when(t < _NTILES)
        def _():
            sv = starts_v[pl.ds(t, 16)]
            r_lo = sv[0]
            r_hi = sv[1]
            r0 = lax.bitwise_and(r_lo, jnp.int32(-8))
            nchunks = (r_hi - r0 + (_CHUNK - 1)) // _CHUNK
            seg_lo = t * _SEG_TILE

            def chunk_base(k):
                return pl.multiple_of(
                    jnp.minimum(r0 + k * _CHUNK, _N - _CHUNK), 8)

            def start_idx(k):
                s = lax.rem(k, _NBUF)
                b = chunk_base(k)
                pltpu.async_copy(eidx.at[pl.ds(b, _CHUNK)], idx_v.at[s],
                                 isem.at[s])
                pltpu.async_copy(sidx.at[pl.ds(b, _CHUNK)], seg_v.at[s],
                                 ssem.at[s])

            def wait_scatter(k):
                s = lax.rem(k, _NBUF)
                s2 = lax.rem(k, 2)
                pltpu.make_async_copy(rows_v.at[s2],
                                      acc_sh.at[segloc_v.at[s]],
                                      csem.at[s2]).wait()
                pltpu.make_async_copy(ones_b.at[pl.ds(0, _CHUNK)],
                                      cnt_sh.at[segloc_v.at[s]],
                                      csem.at[s2]).wait()

            def start_gather(k):
                s = lax.rem(k, _NBUF)
                s2 = lax.rem(k, 2)
                b = chunk_base(k)

                @pl.when(k >= 2)
                def _():
                    wait_scatter(k - 2)

                pltpu.make_async_copy(eidx.at[pl.ds(b, _CHUNK)],
                                      idx_v.at[s], isem.at[s]).wait()
                pltpu.async_copy(table.at[idx_v.at[s]], rows_v.at[s2],
                                 gsem.at[s2])

            @pl.when(nchunks > 0)
            def _():
                start_idx(0)

                @pl.when(nchunks > 1)
                def _():
                    start_idx(1)

            fire_zero(base_row)

            @pl.when(nchunks > 0)
            def _():
                start_gather(0)

            @pl.when(i > 0)
            def _():
                finalize(t - _NW, prev_base)

            drain_zero(base_row)

            def chunk_body(k, _):
                s = lax.rem(k, _NBUF)
                s2 = lax.rem(k, 2)
                b = chunk_base(k)
                shift = (r0 + k * _CHUNK) - b

                @pl.when(k + 1 < nchunks)
                def _():
                    start_gather(k + 1)

                @pl.when(k + 2 < nchunks)
                def _():
                    start_idx(k + 2)

                pltpu.make_async_copy(sidx.at[pl.ds(b, _CHUNK)],
                                      seg_v.at[s], ssem.at[s]).wait()
                for g in range(_CHUNK // 16):
                    sv16 = seg_v[s, pl.ds(g * 16, 16)]
                    ls = sv16 - seg_lo
                    jv = lax.iota(jnp.int32, 16) + (g * 16)
                    lsc = jnp.where(
                        (ls >= 0) & (ls < _SEG_TILE) & (jv >= shift),
                        ls, _SEG_TILE)
                    segloc_v[s, pl.ds(g * 16, 16)] = lsc + base_row

                pltpu.make_async_copy(table.at[idx_v.at[s]], rows_v.at[s2],
                                      gsem.at[s2]).wait()
                pltpu.async_copy(rows_v.at[s2],
                                acc_sh.at[segloc_v.at[s]], csem.at[s2],
                                add=True)
                pltpu.async_copy(ones_b.at[pl.ds(0, _CHUNK)],
                                cnt_sh.at[segloc_v.at[s]], csem.at[s2],
                                add=True)
                return 0

            lax.fori_loop(0, nchunks, chunk_body, 0)

            @pl.when(nchunks > 1)
            def _():
                wait_scatter(nchunks - 2)

            @pl.when(nchunks > 0)
            def _():
                wait_scatter(nchunks - 1)

        return 0

    lax.fori_loop(0, _TILES_PER_W, tile_body, 0)

    i_last = (_NTILES - 1 - wid) // _NW
    t_last = wid + i_last * _NW
    finalize(t_last, region0 + lax.rem(i_last, 2) * _ACC_STRIDE)


def kernel(base_embeddings, extract_index, scatter_index):
    bounds = jnp.arange(0, _STARTS_PAD * _SEG_TILE, _SEG_TILE,
                        dtype=jnp.int32)
    starts = jnp.searchsorted(scatter_index, bounds).astype(jnp.int32)

    mesh = plsc.VectorSubcoreMesh(core_axis_name="c", subcore_axis_name="s")
    f = functools.partial(
        pl.kernel,
        mesh=mesh,
        out_type=jax.ShapeDtypeStruct((_NUM_SEGMENTS, _EMBED_DIM),
                                      jnp.float32),
        scratch_types=[
            pltpu.VMEM((_STARTS_PAD,), jnp.int32),
            pltpu.VMEM((_NBUF, _CHUNK), jnp.int32),
            pltpu.VMEM((_NBUF, _CHUNK), jnp.int32),
            pltpu.VMEM((_NBUF, _CHUNK), jnp.int32),
            pltpu.VMEM((2, _CHUNK, _EMBED_DIM), jnp.float32),
            pltpu.VMEM((_SEG_TILE, _EMBED_DIM), jnp.float32),
            pltpu.VMEM((_SEG_TILE, 16), jnp.float32),
            pltpu.VMEM((_CHUNK, _EMBED_DIM), jnp.float32),
            pltpu.VMEM((_CHUNK, 16), jnp.float32),
            pltpu.VMEM((_CHUNK, 16), jnp.float32),
            pltpu.VMEM_SHARED((_NSUB * 2 * _ACC_STRIDE, _EMBED_DIM),
                              jnp.float32),
            pltpu.VMEM_SHARED((_NSUB * 2 * _ACC_STRIDE, 16), jnp.float32),
            pltpu.SemaphoreType.DMA((_NBUF,)),
            pltpu.SemaphoreType.DMA((_NBUF,)),
            pltpu.SemaphoreType.DMA((2,)),
            pltpu.SemaphoreType.DMA((2,)),
            pltpu.SemaphoreType.DMA,
        ],
        compiler_params=pltpu.CompilerParams(use_tc_tiling_on_sc=False),
    )(_sc_kernel)
    return f(base_embeddings, extract_index, scatter_index, starts)
